# Initial kernel scaffold; baseline (speedup 1.0000x reference)
import jax
import jax.numpy as jnp
from jax.experimental import pallas as pl


def kernel(src, colors, feats):
    raise NotImplementedError("write your pallas kernel here")


# speedup vs baseline: 1.2589x; 1.2589x over previous
import jax
import jax.numpy as jnp
from jax.experimental import pallas as pl

B, H, W = 2, 224, 224
K, D = 19, 1024
HW = H * W
KP = 32
PT = 512


def _expand_kernel(src_ref, ckey_ref, featsT_ref, out_ref):
    s = src_ref[0]
    q = (s * 127.5 + 127.5).astype(jnp.int32)
    qkey = q[0:1, :] * 65536 + q[1:2, :] * 256 + q[2:3, :]
    onehot = (ckey_ref[:] == qkey).astype(jnp.float32)
    out_ref[0] = jnp.dot(featsT_ref[:], onehot,
                         preferred_element_type=jnp.float32)


def kernel(src, colors, feats):
    src2 = src.reshape(B, 3, HW)
    c = colors.astype(jnp.int32)
    key = c[:, 0] * 65536 + c[:, 1] * 256 + c[:, 2]
    i = jnp.arange(K)
    dup = (key[None, :] == key[:, None]) & (i[:, None] > i[None, :])
    key = jnp.where(dup.any(axis=1), -1, key)
    ckey = jnp.full((KP, 1), -1, jnp.int32).at[:K, 0].set(key)
    featsT = jnp.zeros((D, KP), jnp.float32).at[:, :K].set(feats.T)

    out = pl.pallas_call(
        _expand_kernel,
        grid=(B, HW // PT),
        in_specs=[
            pl.BlockSpec((1, 3, PT), lambda b, j: (b, 0, j)),
            pl.BlockSpec((KP, 1), lambda b, j: (0, 0)),
            pl.BlockSpec((D, KP), lambda b, j: (0, 0)),
        ],
        out_specs=pl.BlockSpec((1, D, PT), lambda b, j: (b, 0, j)),
        out_shape=jax.ShapeDtypeStruct((B, D, HW), jnp.float32),
    )(src2, ckey, featsT)
    return out.reshape(B, D, H, W)


# speedup vs baseline: 1.3929x; 1.1064x over previous
import jax
import jax.numpy as jnp
from jax.experimental import pallas as pl
from jax.experimental.pallas import tpu as pltpu

B, H, W = 2, 224, 224
K, D = 19, 1024
HW = H * W
KP = 32
PT = 1024


def _expand_kernel(src_ref, ckey_ref, featsT_ref, out_ref):
    s = src_ref[0]
    q = (s * 127.5 + 127.5).astype(jnp.int32)
    qkey = q[0:1, :] * 65536 + q[1:2, :] * 256 + q[2:3, :]
    onehot = (ckey_ref[:] == qkey).astype(jnp.float32)
    out_ref[0] = jnp.dot(featsT_ref[:], onehot,
                         preferred_element_type=jnp.float32)


def kernel(src, colors, feats):
    src2 = src.reshape(B, 3, HW)
    c = colors.astype(jnp.int32)
    key = c[:, 0] * 65536 + c[:, 1] * 256 + c[:, 2]
    i = jnp.arange(K)
    dup = (key[None, :] == key[:, None]) & (i[:, None] > i[None, :])
    key = jnp.where(dup.any(axis=1), -1, key)
    ckey = jnp.full((KP, 1), -1, jnp.int32).at[:K, 0].set(key)
    featsT = jnp.zeros((D, KP), jnp.float32).at[:, :K].set(feats.T)

    out = pl.pallas_call(
        _expand_kernel,
        grid=(B, HW // PT),
        in_specs=[
            pl.BlockSpec((1, 3, PT), lambda b, j: (b, 0, j)),
            pl.BlockSpec((KP, 1), lambda b, j: (0, 0)),
            pl.BlockSpec((D, KP), lambda b, j: (0, 0)),
        ],
        out_specs=pl.BlockSpec((1, D, PT), lambda b, j: (b, 0, j)),
        out_shape=jax.ShapeDtypeStruct((B, D, HW), jnp.float32),
        compiler_params=pltpu.CompilerParams(
            dimension_semantics=("parallel", "parallel")),
    )(src2, ckey, featsT)
    return out.reshape(B, D, H, W)


# speedup vs baseline: 1.4360x; 1.0309x over previous
import jax
import jax.numpy as jnp
from jax.experimental import pallas as pl
from jax.experimental.pallas import tpu as pltpu

B, H, W = 2, 224, 224
K, D = 19, 1024
HW = H * W
KP = 32
PT = 3584


def _expand_kernel(src_ref, ckey_ref, featsT_ref, out_ref):
    s = src_ref[0]
    q = (s * 127.5 + 127.5).astype(jnp.int32)
    qkey = q[0:1, :] * 65536 + q[1:2, :] * 256 + q[2:3, :]
    onehot = (ckey_ref[:] == qkey).astype(jnp.float32)
    out_ref[0] = jnp.dot(featsT_ref[:], onehot,
                         preferred_element_type=jnp.float32)


def kernel(src, colors, feats):
    src2 = src.reshape(B, 3, HW)
    c = colors.astype(jnp.int32)
    key = c[:, 0] * 65536 + c[:, 1] * 256 + c[:, 2]
    i = jnp.arange(K)
    dup = (key[None, :] == key[:, None]) & (i[:, None] > i[None, :])
    key = jnp.where(dup.any(axis=1), -1, key)
    ckey = jnp.full((KP, 1), -1, jnp.int32).at[:K, 0].set(key)
    featsT = jnp.zeros((D, KP), jnp.float32).at[:, :K].set(feats.T)

    out = pl.pallas_call(
        _expand_kernel,
        grid=(B, HW // PT),
        in_specs=[
            pl.BlockSpec((1, 3, PT), lambda b, j: (b, 0, j)),
            pl.BlockSpec((KP, 1), lambda b, j: (0, 0)),
            pl.BlockSpec((D, KP), lambda b, j: (0, 0)),
        ],
        out_specs=pl.BlockSpec((1, D, PT), lambda b, j: (b, 0, j)),
        out_shape=jax.ShapeDtypeStruct((B, D, HW), jnp.float32),
        compiler_params=pltpu.CompilerParams(
            dimension_semantics=("parallel", "parallel")),
    )(src2, ckey, featsT)
    return out.reshape(B, D, H, W)
